# baseline (speedup 1.0000x reference)
import functools

import jax
import jax.numpy as jnp
from jax import lax
from jax.experimental import pallas as pl
from jax.experimental.pallas import tpu as pltpu
from jax.experimental.pallas import tpu_sc as plsc

N0, N1, N2 = 10000, 5000, 2500
E1, E2 = 320000, 160000
D = 128
N1P, N2P = 5120, 2560
NC, NS = 2, 16
NW = NC * NS
CH = 128

F32 = jnp.float32


def _chunks_of(total, step):
    out, off = [], 0
    while off < total:
        n = min(step, total - off)
        out.append((off, n))
        off += n
    return out


def _make_sc_agg(Ep, Np):
    per_w = Ep // NW
    assert per_w * NW == Ep and per_w % CH == 0
    nch = per_w // CH
    assert nch >= 4
    sl = Np // NS
    assert sl * NS == Np and sl % 16 == 0

    @functools.partial(
        pl.kernel,
        out_type=(
            jax.ShapeDtypeStruct((NC, Np, D), F32),
            jax.ShapeDtypeStruct((NC * Np,), F32),
        ),
        mesh=plsc.VectorSubcoreMesh(core_axis_name="c", subcore_axis_name="s"),
        scratch_types=(
            [pltpu.VMEM((2, CH), jnp.int32)] * 4
            + [pltpu.VMEM((CH, D), F32)] * 4
            + [
                pltpu.VMEM((CH,), F32),
                pltpu.VMEM((64, D), F32),
                pltpu.VMEM((Np,), F32),
                pltpu.VMEM_SHARED((Np, D), F32),
                pltpu.VMEM_SHARED((Np,), F32),
            ]
            + [pltpu.SemaphoreType.DMA] * 4
            + [pltpu.SemaphoreType.DMA] * 4
        ),
    )
    def agg(table, ei, sum_out, cnt_out,
            ib0, ib1, ib2, ib3, rb0, rb1, rb2, rb3,
            ones_r, zbuf, cbuf, acc, cnt,
            g0, g1, g2, g3, s0, s1, s2, s3):
        idxb = (ib0, ib1, ib2, ib3)
        rowsb = (rb0, rb1, rb2, rb3)
        gsem = (g0, g1, g2, g3)
        ssem = (s0, s1, s2, s3)
        c = lax.axis_index("c")
        s = lax.axis_index("s")
        wid = c * NS + s
        row0 = s * sl

        z16 = jnp.zeros((16,), F32)
        o16 = jnp.ones((16,), F32)
        for j in range(CH // 16):
            ones_r[pl.ds(j * 16, 16)] = o16

        @pl.loop(0, 64)
        def _zero_rows(i):
            for j in range(D // 16):
                zbuf[i, pl.ds(j * 16, 16)] = z16

        @pl.loop(0, sl // 16)
        def _zero_cnt(k):
            cbuf[pl.ds(k * 16, 16)] = z16

        for off, n in _chunks_of(sl, 64):
            pltpu.sync_copy(zbuf.at[pl.ds(0, n)], acc.at[pl.ds(row0 + off, n)])
        pltpu.sync_copy(cbuf.at[pl.ds(0, sl)], cnt.at[pl.ds(row0, sl)])
        plsc.subcore_barrier()

        cb = wid * nch

        def fire(j, b):
            pltpu.sync_copy(ei.at[cb + j], idxb[b])
            pltpu.async_copy(table.at[idxb[b].at[0]], rowsb[b], gsem[b])

        def gwait(b):
            pltpu.make_async_copy(table.at[idxb[b].at[0]], rowsb[b],
                                  gsem[b]).wait()

        def ascat(b):
            pltpu.async_copy(rowsb[b], acc.at[idxb[b].at[1]], ssem[b],
                             add=True)
            pltpu.async_copy(ones_r, cnt.at[idxb[b].at[1]], ssem[b], add=True)

        def swait(b):
            pltpu.make_async_copy(rowsb[b], acc.at[idxb[b].at[1]],
                                  ssem[b]).wait()
            pltpu.make_async_copy(ones_r, cnt.at[idxb[b].at[1]],
                                  ssem[b]).wait()

        def station(j, jmod):
            b = jmod % 4
            d = j - 2
            if isinstance(j, int):
                if j <= nch - 1:
                    if j >= 4:
                        swait(b)
                    fire(j, b)
                if 0 <= d <= nch - 1:
                    bd = (jmod - 2) % 4
                    gwait(bd)
                    ascat(bd)
            else:
                swait(b)
                fire(j, b)
                bd = (jmod - 2) % 4
                gwait(bd)
                ascat(bd)

        for j in range(0, 4):
            station(j, j)
        nquad = (nch - 4) // 4
        if nquad > 0:
            @pl.loop(0, nquad)
            def _quads(q):
                for k in range(4):
                    station(4 + 4 * q + k, 4 + k)
        for j in range(4 + 4 * nquad, nch + 2):
            station(j, j)
        for jj in range(max(nch - 4, 0), nch):
            swait(jj % 4)

        plsc.subcore_barrier()

        for off, n in _chunks_of(sl, 64):
            pltpu.sync_copy(acc.at[pl.ds(row0 + off, n)], zbuf.at[pl.ds(0, n)])
            pltpu.sync_copy(zbuf.at[pl.ds(0, n)],
                            sum_out.at[c, pl.ds(row0 + off, n)])

        @pl.when(s == 0)
        def _write_cnt():
            pltpu.sync_copy(cnt, cbuf)
            pltpu.sync_copy(cbuf, cnt_out.at[pl.ds(c * Np, Np)])

    return agg


def _make_tc_dense(Np, act):
    blk = 512
    grid = Np // blk
    dn = (((1,), (1,)), ((), ()))

    def body(p_ref0, p_ref1, cnt_ref, x_ref, wl_ref, wr_ref, b_ref, o_ref):
        i = pl.program_id(0)
        ssum = p_ref0[0] + p_ref1[0]
        cb = cnt_ref[:, pl.ds(i * blk, blk)]
        csum = jnp.maximum(cb[0] + cb[1], 1.0)
        mean = ssum * (1.0 / csum)[:, None]
        h = (lax.dot_general(mean, wl_ref[...], dn, preferred_element_type=F32)
             + lax.dot_general(x_ref[...], wr_ref[...], dn,
                               preferred_element_type=F32)
             + b_ref[...])
        if act == "relu":
            h = jnp.maximum(h, 0.0)
        else:
            m = jnp.max(h, axis=1, keepdims=True)
            e = jnp.exp(h - m)
            h = h - m - jnp.log(jnp.sum(e, axis=1, keepdims=True))
        o_ref[...] = h

    return pl.pallas_call(
        body,
        grid=(grid,),
        in_specs=[
            pl.BlockSpec((1, blk, D), lambda i: (0, i, 0)),
            pl.BlockSpec((1, blk, D), lambda i: (1, i, 0)),
            pl.BlockSpec((NC, Np), lambda i: (0, 0)),
            pl.BlockSpec((blk, D), lambda i: (i, 0)),
            pl.BlockSpec((D, D), lambda i: (0, 0)),
            pl.BlockSpec((D, D), lambda i: (0, 0)),
            pl.BlockSpec((1, D), lambda i: (0, 0)),
        ],
        out_specs=pl.BlockSpec((blk, D), lambda i: (i, 0)),
        out_shape=jax.ShapeDtypeStruct((Np, D), F32),
    )


def _round_up(v, m):
    return -(-v // m) * m


def _chunk_pack(edge_index, E, Ep, n_real, n_pad):
    pad = Ep - E
    src = jnp.concatenate([edge_index[0], jnp.zeros((pad,), jnp.int32)])
    spread = n_real + jnp.arange(pad, dtype=jnp.int32) % (n_pad - n_real)
    dst = jnp.concatenate([edge_index[1], spread])
    return jnp.stack([src.reshape(-1, CH), dst.reshape(-1, CH)], axis=1)


E1P = _round_up(E1, NW * CH)
E2P = _round_up(E2, NW * CH)

_agg1 = _make_sc_agg(E1P, N1P)
_agg2 = _make_sc_agg(E2P, N2P)
_dense1 = _make_tc_dense(N1P, "relu")
_dense2 = _make_tc_dense(N2P, "logsoftmax")


def kernel(x, edge_index1, edge_index2, W1l, b1l, W1r, W2l, b2l, W2r):
    ei1 = _chunk_pack(edge_index1, E1, E1P, N1, N1P)
    ei2 = _chunk_pack(edge_index2, E2, E2P, N2, N2P)
    b1 = jnp.reshape(b1l, (1, D))
    b2 = jnp.reshape(b2l, (1, D))

    sum1, cnt1 = _agg1(x, ei1)
    h = _dense1(sum1, sum1, cnt1.reshape(NC, N1P), x, W1l, W1r, b1)
    sum2, cnt2 = _agg2(h, ei2)
    out = _dense2(sum2, sum2, cnt2.reshape(NC, N2P), h, W2l, W2r, b2)
    return out[:N2]


# speedup vs baseline: 2.8012x; 2.8012x over previous
import functools

import jax
import jax.numpy as jnp
from jax import lax
from jax.experimental import pallas as pl
from jax.experimental.pallas import tpu as pltpu
from jax.experimental.pallas import tpu_sc as plsc

N0, N1, N2 = 10000, 5000, 2500
E1, E2 = 320000, 160000
D = 128
N1P, N2P = 5120, 2560
NC, NS = 2, 16
NW = NC * NS
CH = 128

F32 = jnp.float32


def _chunks_of(total, step):
    out, off = [], 0
    while off < total:
        n = min(step, total - off)
        out.append((off, n))
        off += n
    return out


def _make_sc_agg(n_table, E, Np):
    per_w = E // NW
    assert per_w * NW == E
    nch = per_w // CH
    tail = per_w - nch * CH
    assert nch >= 4
    sl = Np // NS
    assert sl * NS == Np and sl % 16 == 0

    @functools.partial(
        pl.kernel,
        out_type=(
            jax.ShapeDtypeStruct((NC, Np, D), F32),
            jax.ShapeDtypeStruct((NC * Np,), F32),
        ),
        mesh=plsc.VectorSubcoreMesh(core_axis_name="c", subcore_axis_name="s"),
        scratch_types=(
            [pltpu.VMEM((CH,), jnp.int32)] * 8
            + [pltpu.VMEM((CH,), jnp.int32)] * 8
            + [pltpu.VMEM((CH, D), F32)] * 4
            + [
                pltpu.VMEM((max(tail, 8),), jnp.int32),
                pltpu.VMEM((max(tail, 8),), jnp.int32),
                pltpu.VMEM((CH,), F32),
                pltpu.VMEM((64, D), F32),
                pltpu.VMEM((Np,), F32),
                pltpu.VMEM_SHARED((Np, D), F32),
                pltpu.VMEM_SHARED((Np,), F32),
            ]
            + [pltpu.SemaphoreType.DMA] * 4
            + [pltpu.SemaphoreType.DMA] * 4
            + [pltpu.SemaphoreType.DMA] * 4
        ),
    )
    def agg(table, src, dst, sum_out, cnt_out,
            sb0, sb1, sb2, sb3, sb4, sb5, sb6, sb7,
            db0, db1, db2, db3, db4, db5, db6, db7,
            rb0, rb1, rb2, rb3,
            src_t, dst_t, ones_r, zbuf, cbuf, acc, cnt,
            i0, i1, i2, i3, g0, g1, g2, g3, s0, s1, s2, s3):
        srcb = (sb0, sb1, sb2, sb3, sb4, sb5, sb6, sb7)
        dstb = (db0, db1, db2, db3, db4, db5, db6, db7)
        rowsb = (rb0, rb1, rb2, rb3)
        isem = (i0, i1, i2, i3)
        gsem = (g0, g1, g2, g3)
        ssem = (s0, s1, s2, s3)
        c = lax.axis_index("c")
        s = lax.axis_index("s")
        wid = c * NS + s
        ebase = wid * per_w
        row0 = s * sl

        z16 = jnp.zeros((16,), F32)
        o16 = jnp.ones((16,), F32)
        for j in range(CH // 16):
            ones_r[pl.ds(j * 16, 16)] = o16

        @pl.loop(0, 64)
        def _zero_rows(i):
            for j in range(D // 16):
                zbuf[i, pl.ds(j * 16, 16)] = z16

        @pl.loop(0, sl // 16)
        def _zero_cnt(k):
            cbuf[pl.ds(k * 16, 16)] = z16

        for off, n in _chunks_of(sl, 64):
            pltpu.sync_copy(zbuf.at[pl.ds(0, n)], acc.at[pl.ds(row0 + off, n)])
        pltpu.sync_copy(cbuf.at[pl.ds(0, sl)], cnt.at[pl.ds(row0, sl)])
        plsc.subcore_barrier()

        def fire_idx(j, jm):
            base = ebase + j * CH
            pltpu.async_copy(src.at[pl.ds(base, CH)], srcb[jm % 8],
                             isem[jm % 4])
            pltpu.async_copy(dst.at[pl.ds(base, CH)], dstb[jm % 8],
                             isem[jm % 4])

        def iwait(jm):
            pltpu.make_async_copy(src.at[pl.ds(0, CH)], srcb[jm % 8],
                                  isem[jm % 4]).wait()
            pltpu.make_async_copy(dst.at[pl.ds(0, CH)], dstb[jm % 8],
                                  isem[jm % 4]).wait()

        def fire_g(jm):
            pltpu.async_copy(table.at[srcb[jm % 8]], rowsb[jm % 4],
                             gsem[jm % 4])

        def gwait(jm):
            pltpu.make_async_copy(table.at[srcb[jm % 8]], rowsb[jm % 4],
                                  gsem[jm % 4]).wait()

        def ascat(jm):
            pltpu.async_copy(rowsb[jm % 4], acc.at[dstb[jm % 8]],
                             ssem[jm % 4], add=True)
            pltpu.async_copy(ones_r, cnt.at[dstb[jm % 8]], ssem[jm % 4],
                             add=True)

        def swait(jm):
            pltpu.make_async_copy(rowsb[jm % 4], acc.at[dstb[jm % 8]],
                                  ssem[jm % 4]).wait()
            pltpu.make_async_copy(ones_r, cnt.at[dstb[jm % 8]],
                                  ssem[jm % 4]).wait()

        def station(j, jm):
            py = isinstance(j, int)
            if not py or 4 <= j <= nch - 1:
                swait(jm - 4)
            if not py or j + 2 <= nch - 1:
                fire_idx(j + 2, jm + 2)
            if not py or j <= nch - 1:
                iwait(jm)
                fire_g(jm)
            if not py or 2 <= j <= nch + 1:
                gwait(jm - 2)
                ascat(jm - 2)

        fire_idx(0, 0)
        fire_idx(1, 1)
        for j in range(0, 8):
            station(j, j)
        noct = (nch - 10) // 8
        if noct > 0:
            @pl.loop(0, noct)
            def _octs(q):
                for k in range(8):
                    station(8 + 8 * q + k, 8 + k)
        for j in range(8 + 8 * max(noct, 0), nch + 2):
            station(j, j)
        for jj in range(max(nch - 4, 0), nch):
            swait(jj)

        if tail:
            base = ebase + nch * CH
            pltpu.sync_copy(src.at[pl.ds(base, tail)], src_t)
            gd = pltpu.async_copy(table.at[src_t], rb0.at[pl.ds(0, tail)],
                                  g0)
            pltpu.sync_copy(dst.at[pl.ds(base, tail)], dst_t)
            pltpu.sync_copy(ones_r.at[pl.ds(0, tail)], cnt.at[dst_t], add=True)
            gd.wait()
            pltpu.sync_copy(rb0.at[pl.ds(0, tail)], acc.at[dst_t], add=True)

        plsc.subcore_barrier()

        for off, n in _chunks_of(sl, 64):
            pltpu.sync_copy(acc.at[pl.ds(row0 + off, n)], zbuf.at[pl.ds(0, n)])
            pltpu.sync_copy(zbuf.at[pl.ds(0, n)],
                            sum_out.at[c, pl.ds(row0 + off, n)])

        @pl.when(s == 0)
        def _write_cnt():
            pltpu.sync_copy(cnt, cbuf)
            pltpu.sync_copy(cbuf, cnt_out.at[pl.ds(c * Np, Np)])

    return agg


def _make_tc_dense(Np, act):
    blk = 512
    grid = Np // blk
    dn = (((1,), (1,)), ((), ()))

    def body(p_ref0, p_ref1, cnt_ref, x_ref, wl_ref, wr_ref, b_ref, o_ref):
        i = pl.program_id(0)
        ssum = p_ref0[0] + p_ref1[0]
        cb = cnt_ref[:, pl.ds(i * blk, blk)]
        csum = jnp.maximum(cb[0] + cb[1], 1.0)
        mean = ssum * (1.0 / csum)[:, None]
        h = (lax.dot_general(mean, wl_ref[...], dn, preferred_element_type=F32)
             + lax.dot_general(x_ref[...], wr_ref[...], dn,
                               preferred_element_type=F32)
             + b_ref[...])
        if act == "relu":
            h = jnp.maximum(h, 0.0)
        else:
            m = jnp.max(h, axis=1, keepdims=True)
            e = jnp.exp(h - m)
            h = h - m - jnp.log(jnp.sum(e, axis=1, keepdims=True))
        o_ref[...] = h

    return pl.pallas_call(
        body,
        grid=(grid,),
        in_specs=[
            pl.BlockSpec((1, blk, D), lambda i: (0, i, 0)),
            pl.BlockSpec((1, blk, D), lambda i: (1, i, 0)),
            pl.BlockSpec((NC, Np), lambda i: (0, 0)),
            pl.BlockSpec((blk, D), lambda i: (i, 0)),
            pl.BlockSpec((D, D), lambda i: (0, 0)),
            pl.BlockSpec((D, D), lambda i: (0, 0)),
            pl.BlockSpec((1, D), lambda i: (0, 0)),
        ],
        out_specs=pl.BlockSpec((blk, D), lambda i: (i, 0)),
        out_shape=jax.ShapeDtypeStruct((Np, D), F32),
    )


_agg1 = _make_sc_agg(N0, E1, N1P)
_agg2 = _make_sc_agg(N1P, E2, N2P)
_dense1 = _make_tc_dense(N1P, "relu")
_dense2 = _make_tc_dense(N2P, "logsoftmax")


def kernel(x, edge_index1, edge_index2, W1l, b1l, W1r, W2l, b2l, W2r):
    src1, dst1 = edge_index1[0], edge_index1[1]
    src2, dst2 = edge_index2[0], edge_index2[1]
    b1 = jnp.reshape(b1l, (1, D))
    b2 = jnp.reshape(b2l, (1, D))

    sum1, cnt1 = _agg1(x, src1, dst1)
    h = _dense1(sum1, sum1, cnt1.reshape(NC, N1P), x[:N1P], W1l, W1r, b1)
    sum2, cnt2 = _agg2(h, src2, dst2)
    out = _dense2(sum2, sum2, cnt2.reshape(NC, N2P), h[:N2P], W2l, W2r, b2)
    return out[:N2]


# speedup vs baseline: 2.8450x; 1.0157x over previous
import functools

import jax
import jax.numpy as jnp
from jax import lax
from jax.experimental import pallas as pl
from jax.experimental.pallas import tpu as pltpu
from jax.experimental.pallas import tpu_sc as plsc

N0, N1, N2 = 10000, 5000, 2500
E1, E2 = 320000, 160000
D = 128
N1P, N2P = 5120, 2560
NC, NS = 2, 16
NW = NC * NS
CH = 128

F32 = jnp.float32


def _chunks_of(total, step):
    out, off = [], 0
    while off < total:
        n = min(step, total - off)
        out.append((off, n))
        off += n
    return out


def _make_sc_agg(n_table, E, Np):
    per_w = E // NW
    assert per_w * NW == E
    nch = per_w // CH
    tail = per_w - nch * CH
    assert nch >= 4
    sl = Np // NS
    assert sl * NS == Np and sl % 16 == 0

    @functools.partial(
        pl.kernel,
        out_type=(
            jax.ShapeDtypeStruct((NC, Np, D), F32),
            jax.ShapeDtypeStruct((NC * Np,), F32),
        ),
        mesh=plsc.VectorSubcoreMesh(core_axis_name="c", subcore_axis_name="s"),
        scratch_types=(
            [pltpu.VMEM((CH,), jnp.int32)] * 8
            + [pltpu.VMEM((CH,), jnp.int32)] * 8
            + [pltpu.VMEM((CH, D), F32)] * 4
            + [
                pltpu.VMEM((max(tail, 8),), jnp.int32),
                pltpu.VMEM((max(tail, 8),), jnp.int32),
                pltpu.VMEM((CH,), F32),
                pltpu.VMEM((64, D), F32),
                pltpu.VMEM((Np,), F32),
                pltpu.VMEM_SHARED((Np, D), F32),
                pltpu.VMEM_SHARED((Np,), F32),
            ]
            + [pltpu.SemaphoreType.DMA] * 4
            + [pltpu.SemaphoreType.DMA] * 4
            + [pltpu.SemaphoreType.DMA] * 4
        ),
    )
    def agg(table, src, dst, sum_out, cnt_out,
            sb0, sb1, sb2, sb3, sb4, sb5, sb6, sb7,
            db0, db1, db2, db3, db4, db5, db6, db7,
            rb0, rb1, rb2, rb3,
            src_t, dst_t, ones_r, zbuf, cbuf, acc, cnt,
            i0, i1, i2, i3, g0, g1, g2, g3, s0, s1, s2, s3):
        srcb = (sb0, sb1, sb2, sb3, sb4, sb5, sb6, sb7)
        dstb = (db0, db1, db2, db3, db4, db5, db6, db7)
        rowsb = (rb0, rb1, rb2, rb3)
        isem = (i0, i1, i2, i3)
        gsem = (g0, g1, g2, g3)
        ssem = (s0, s1, s2, s3)
        c = lax.axis_index("c")
        s = lax.axis_index("s")
        wid = c * NS + s
        ebase = wid * per_w
        row0 = s * sl

        z16 = jnp.zeros((16,), F32)
        o16 = jnp.ones((16,), F32)
        for j in range(CH // 16):
            ones_r[pl.ds(j * 16, 16)] = o16

        @pl.loop(0, 64)
        def _zero_rows(i):
            for j in range(D // 16):
                zbuf[i, pl.ds(j * 16, 16)] = z16

        @pl.loop(0, sl // 16)
        def _zero_cnt(k):
            cbuf[pl.ds(k * 16, 16)] = z16

        for off, n in _chunks_of(sl, 64):
            pltpu.sync_copy(zbuf.at[pl.ds(0, n)], acc.at[pl.ds(row0 + off, n)])
        pltpu.sync_copy(cbuf.at[pl.ds(0, sl)], cnt.at[pl.ds(row0, sl)])
        plsc.subcore_barrier()

        def fire_idx(j, jm):
            base = ebase + j * CH
            pltpu.async_copy(src.at[pl.ds(base, CH)], srcb[jm % 8],
                             isem[jm % 4])
            pltpu.async_copy(dst.at[pl.ds(base, CH)], dstb[jm % 8],
                             isem[jm % 4])

        def iwait(jm):
            pltpu.make_async_copy(src.at[pl.ds(0, CH)], srcb[jm % 8],
                                  isem[jm % 4]).wait()
            pltpu.make_async_copy(dst.at[pl.ds(0, CH)], dstb[jm % 8],
                                  isem[jm % 4]).wait()

        def fire_g(jm):
            pltpu.async_copy(table.at[srcb[jm % 8]], rowsb[jm % 4],
                             gsem[jm % 4])

        def gwait(jm):
            pltpu.make_async_copy(table.at[srcb[jm % 8]], rowsb[jm % 4],
                                  gsem[jm % 4]).wait()

        def ascat(jm):
            pltpu.async_copy(rowsb[jm % 4], acc.at[dstb[jm % 8]],
                             ssem[jm % 4], add=True)
            pltpu.async_copy(ones_r, cnt.at[dstb[jm % 8]], ssem[jm % 4],
                             add=True)

        def swait(jm):
            pltpu.make_async_copy(rowsb[jm % 4], acc.at[dstb[jm % 8]],
                                  ssem[jm % 4]).wait()
            pltpu.make_async_copy(ones_r, cnt.at[dstb[jm % 8]],
                                  ssem[jm % 4]).wait()

        def station(j, jm):
            py = isinstance(j, int)
            if not py or 4 <= j <= nch - 1:
                swait(jm - 4)
            if not py or j + 2 <= nch - 1:
                fire_idx(j + 2, jm + 2)
            if not py or j <= nch - 1:
                iwait(jm)
                fire_g(jm)
            if not py or 2 <= j <= nch + 1:
                gwait(jm - 2)
                ascat(jm - 2)

        fire_idx(0, 0)
        fire_idx(1, 1)
        for j in range(0, 8):
            station(j, j)
        noct = (nch - 10) // 8
        if noct > 0:
            @pl.loop(0, noct)
            def _octs(q):
                for k in range(8):
                    station(8 + 8 * q + k, 8 + k)
        for j in range(8 + 8 * max(noct, 0), nch + 2):
            station(j, j)
        for jj in range(max(nch - 4, 0), nch):
            swait(jj)

        if tail:
            base = ebase + nch * CH
            pltpu.sync_copy(src.at[pl.ds(base, tail)], src_t)
            gd = pltpu.async_copy(table.at[src_t], rb0.at[pl.ds(0, tail)],
                                  g0)
            pltpu.sync_copy(dst.at[pl.ds(base, tail)], dst_t)
            pltpu.sync_copy(ones_r.at[pl.ds(0, tail)], cnt.at[dst_t], add=True)
            gd.wait()
            pltpu.sync_copy(rb0.at[pl.ds(0, tail)], acc.at[dst_t], add=True)

        plsc.subcore_barrier()

        for off, n in _chunks_of(sl, 64):
            pltpu.sync_copy(acc.at[pl.ds(row0 + off, n)], zbuf.at[pl.ds(0, n)])
            pltpu.sync_copy(zbuf.at[pl.ds(0, n)],
                            sum_out.at[c, pl.ds(row0 + off, n)])

        @pl.when(s == 0)
        def _write_cnt():
            pltpu.sync_copy(cnt, cbuf)
            pltpu.sync_copy(cbuf, cnt_out.at[pl.ds(c * Np, Np)])

    return agg


def _make_tc_dense(Np, rows_out, act):
    blk = 512
    grid = -(-rows_out // blk)
    dn = (((1,), (1,)), ((), ()))

    def body(p_ref0, p_ref1, cnt_ref, x_ref, wl_ref, wr_ref, b_ref, o_ref):
        i = pl.program_id(0)
        ssum = p_ref0[0] + p_ref1[0]
        cb = cnt_ref[:, pl.ds(i * blk, blk)]
        csum = jnp.maximum(cb[0] + cb[1], 1.0)
        mean = ssum * (1.0 / csum)[:, None]
        h = (lax.dot_general(mean, wl_ref[...], dn, preferred_element_type=F32)
             + lax.dot_general(x_ref[...], wr_ref[...], dn,
                               preferred_element_type=F32)
             + b_ref[...])
        if act == "relu":
            h = jnp.maximum(h, 0.0)
        else:
            m = jnp.max(h, axis=1, keepdims=True)
            e = jnp.exp(h - m)
            h = h - m - jnp.log(jnp.sum(e, axis=1, keepdims=True))
        o_ref[...] = h

    return pl.pallas_call(
        body,
        grid=(grid,),
        in_specs=[
            pl.BlockSpec((1, blk, D), lambda i: (0, i, 0)),
            pl.BlockSpec((1, blk, D), lambda i: (1, i, 0)),
            pl.BlockSpec((NC, Np), lambda i: (0, 0)),
            pl.BlockSpec((blk, D), lambda i: (i, 0)),
            pl.BlockSpec((D, D), lambda i: (0, 0)),
            pl.BlockSpec((D, D), lambda i: (0, 0)),
            pl.BlockSpec((1, D), lambda i: (0, 0)),
        ],
        out_specs=pl.BlockSpec((blk, D), lambda i: (i, 0)),
        out_shape=jax.ShapeDtypeStruct((rows_out, D), F32),
    )


_agg1 = _make_sc_agg(N0, E1, N1P)
_agg2 = _make_sc_agg(N1P, E2, N2P)
_dense1 = _make_tc_dense(N1P, N1P, "relu")
_dense2 = _make_tc_dense(N2P, N2, "logsoftmax")


def kernel(x, edge_index1, edge_index2, W1l, b1l, W1r, W2l, b2l, W2r):
    src1, dst1 = edge_index1[0], edge_index1[1]
    src2, dst2 = edge_index2[0], edge_index2[1]
    b1 = jnp.reshape(b1l, (1, D))
    b2 = jnp.reshape(b2l, (1, D))

    sum1, cnt1 = _agg1(x, src1, dst1)
    h = _dense1(sum1, sum1, cnt1.reshape(NC, N1P), x, W1l, W1r, b1)
    sum2, cnt2 = _agg2(h, src2, dst2)
    return _dense2(sum2, sum2, cnt2.reshape(NC, N2P), h, W2l, W2r, b2)
